# baseline (speedup 1.0000x reference)
import jax
import jax.numpy as jnp
from jax.experimental import pallas as pl

_N = 65536
_C = 82
_K_POS = 256
_K_NEG = 512
_BLK = 1024
_INTERPRET = False


def _digamma(x):
    acc = jnp.zeros_like(x)
    w = x
    for _ in range(6):
        small = w < 7.0
        acc = acc + jnp.where(small, 1.0 / w, 0.0)
        w = jnp.where(small, w + 1.0, w)
    inv = 1.0 / w
    inv2 = inv * inv
    s = jnp.log(w) - 0.5 * inv - inv2 * (
        (1.0 / 12.0) - inv2 * ((1.0 / 120.0) - inv2 * (1.0 / 252.0)))
    return s - acc


def _sort_key(x):
    u = jax.lax.bitcast_convert_type(x, jnp.uint32)
    sign = u >> jnp.uint32(31)
    flip = sign * jnp.uint32(0x7FFFFFFF) + jnp.uint32(0x80000000)
    return u ^ flip


def _pass1_body(scores_ref, labels_ref, obj_ref, kp_ref, kn_ref, ff_ref, fb_ref):
    s = scores_ref[...]
    lab = labels_ref[...]
    obj = obj_ref[...]
    E = jnp.exp(s)
    T = jnp.sum(E, axis=1, keepdims=True) + float(_C)
    cols = jax.lax.broadcasted_iota(jnp.int32, s.shape, 1)
    E_l = jnp.sum(jnp.where(cols == lab, E, 0.0), axis=1, keepdims=True)
    E_lm1 = jnp.sum(jnp.where(cols == lab - 1, E, 0.0), axis=1, keepdims=True)
    E79 = E[:, 79:80]
    E80 = E[:, 80:81]
    E81 = E[:, 81:82]
    m80 = jnp.max(s[:, :80], axis=1, keepdims=True)

    A = _digamma(T - E_l - 1.0)
    B = _digamma(T - E80 - 1.0)
    le79 = lab <= 79
    E_t = jnp.where(le79, E80, E79)
    E_g = jnp.where(le79, E_l, E81)
    ff = (1.0 - obj) * (A - _digamma(E_t + 1.0)) + jnp.where(
        lab != 81, obj * (B - _digamma(E_g + 1.0)), 0.0)
    x_t = jnp.where(lab == 81, E80, E81)
    x_g = jnp.where((lab >= 1) & (lab <= 80), E_lm1, E81)
    fb = obj * (A - _digamma(x_t + 1.0)) + 0.2 * (1.0 - obj) * (
        B - _digamma(x_g + 1.0))

    fg = lab != 81
    pos = jnp.where(fg, -m80, -jnp.inf)
    neg = jnp.where(fg, -jnp.inf, -s[:, 81:82])
    kp_ref[...] = _sort_key(pos)
    kn_ref[...] = _sort_key(neg)
    ff_ref[...] = ff
    fb_ref[...] = fb


def _select_sum(keys, ik, vals, k):
    def hi_body(i, a):
        b = (jnp.int32(31) - i).astype(jnp.uint32)
        trial = a | (jnp.uint32(1) << b)
        cnt = jnp.sum((keys >= trial).astype(jnp.int32))
        return jnp.where(cnt >= k, trial, a)

    a_hi = jax.lax.fori_loop(0, 32, hi_body, jnp.uint32(0))
    gt = keys > a_hi
    eq = keys == a_hi
    cnt_gt = jnp.sum(gt.astype(jnp.int32))

    def lo_body(i, a):
        b = (jnp.int32(15) - i).astype(jnp.uint32)
        trial = a | (jnp.uint32(1) << b)
        cnt = cnt_gt + jnp.sum((eq & (ik >= trial)).astype(jnp.int32))
        return jnp.where(cnt >= k, trial, a)

    a_lo = jax.lax.fori_loop(0, 16, lo_body, jnp.uint32(0))
    sel = gt | (eq & (ik >= a_lo))
    return jnp.sum(jnp.where(sel, vals, 0.0))


def _pass2_body(kp_ref, kn_ref, ff_ref, fb_ref, out_ref):
    kp = kp_ref[...]
    kn = kn_ref[...]
    r = jax.lax.broadcasted_iota(jnp.uint32, kp.shape, 0)
    c = jax.lax.broadcasted_iota(jnp.uint32, kp.shape, 1)
    ik = jnp.uint32(_N - 1) - (r * jnp.uint32(kp.shape[1]) + c)
    s_pos = _select_sum(kp, ik, ff_ref[...], _K_POS)
    s_neg = _select_sum(kn, ik, fb_ref[...], _K_NEG)
    out_ref[...] = jnp.full((1, 1), (s_pos + s_neg) / float(_K_POS + _K_NEG))


def kernel(scores, labels, squarescores, objectness, ious):
    del squarescores, ious
    lab2 = labels.reshape(_N, 1)
    obj2 = objectness.reshape(_N, 1)
    nblk = _N // _BLK
    col = jax.ShapeDtypeStruct((_N, 1), jnp.float32)
    colu = jax.ShapeDtypeStruct((_N, 1), jnp.uint32)
    kp, kn, ff, fb = pl.pallas_call(
        _pass1_body,
        grid=(nblk,),
        in_specs=[
            pl.BlockSpec((_BLK, _C), lambda i: (i, 0)),
            pl.BlockSpec((_BLK, 1), lambda i: (i, 0)),
            pl.BlockSpec((_BLK, 1), lambda i: (i, 0)),
        ],
        out_specs=[pl.BlockSpec((_BLK, 1), lambda i: (i, 0))] * 4,
        out_shape=[colu, colu, col, col],
        interpret=_INTERPRET,
    )(scores, lab2, obj2)

    shape2 = (_N // 128, 128)
    kp = kp.reshape(shape2)
    kn = kn.reshape(shape2)
    ff = ff.reshape(shape2)
    fb = fb.reshape(shape2)
    out = pl.pallas_call(
        _pass2_body,
        out_shape=jax.ShapeDtypeStruct((1, 1), jnp.float32),
        interpret=_INTERPRET,
    )(kp, kn, ff, fb)
    return out[0, 0]


# speedup vs baseline: 1.6703x; 1.6703x over previous
import jax
import jax.numpy as jnp
from jax.experimental import pallas as pl

_N = 65536
_C = 82
_K_POS = 256
_K_NEG = 512
_BLK = 1024
_INTERPRET = False


def _digamma_large(x):
    inv = 1.0 / x
    inv2 = inv * inv
    return jnp.log(x) - 0.5 * inv - inv2 * (
        (1.0 / 12.0) - inv2 * ((1.0 / 120.0) - inv2 * (1.0 / 252.0)))


def _digamma_small(x):
    q = ((((x + 15.0) * x + 85.0) * x + 225.0) * x + 274.0) * x * x + 120.0 * x
    qp = ((((6.0 * x + 75.0) * x + 340.0) * x + 675.0) * x + 548.0) * x + 120.0
    return _digamma_large(x + 6.0) - qp / q


def _sort_key(x):
    u = jax.lax.bitcast_convert_type(x, jnp.uint32)
    sign = u >> jnp.uint32(31)
    flip = sign * jnp.uint32(0x7FFFFFFF) + jnp.uint32(0x80000000)
    return u ^ flip


def _pass1_body(scores_ref, labels_ref, kp_ref, kn_ref, t_ref, sl_ref,
                slm1_ref, s79_ref, s80_ref, s81_ref):
    s = scores_ref[...]
    lab = labels_ref[...]
    E = jnp.exp(s)
    t_ref[...] = jnp.sum(E, axis=1, keepdims=True)
    cols = jax.lax.broadcasted_iota(jnp.int32, s.shape, 1)
    sl_ref[...] = jnp.sum(jnp.where(cols == lab, s, 0.0), axis=1,
                          keepdims=True)
    slm1_ref[...] = jnp.sum(jnp.where(cols == lab - 1, s, 0.0), axis=1,
                            keepdims=True)
    s79_ref[...] = s[:, 79:80]
    s80_ref[...] = s[:, 80:81]
    s81_ref[...] = s[:, 81:82]
    m80 = jnp.max(s[:, :80], axis=1, keepdims=True)
    fg = lab != 81
    pos = jnp.where(fg, -m80, -jnp.inf)
    neg = jnp.where(fg, -jnp.inf, -s[:, 81:82])
    kp_ref[...] = _sort_key(pos)
    kn_ref[...] = _sort_key(neg)


def _select_sum(keys, ik, vals, k):
    def hi_body(i, a):
        b = (jnp.int32(31) - i).astype(jnp.uint32)
        trial = a | (jnp.uint32(1) << b)
        cnt = jnp.sum((keys >= trial).astype(jnp.int32))
        return jnp.where(cnt >= k, trial, a)

    a_hi = jax.lax.fori_loop(0, 32, hi_body, jnp.uint32(0))
    gt = keys > a_hi
    eq = keys == a_hi
    cnt_gt = jnp.sum(gt.astype(jnp.int32))

    def lo_body(i, a):
        b = (jnp.int32(15) - i).astype(jnp.uint32)
        trial = a | (jnp.uint32(1) << b)
        cnt = cnt_gt + jnp.sum((eq & (ik >= trial)).astype(jnp.int32))
        return jnp.where(cnt >= k, trial, a)

    a_lo = jax.lax.fori_loop(0, 16, lo_body, jnp.uint32(0))
    sel = gt | (eq & (ik >= a_lo))
    return jnp.sum(jnp.where(sel, vals, 0.0))


def _pass2_body(kp_ref, kn_ref, t_ref, sl_ref, slm1_ref, s79_ref, s80_ref,
                s81_ref, lab_ref, obj_ref, out_ref):
    lab = lab_ref[...]
    obj = obj_ref[...]
    T = t_ref[...] + float(_C)
    E_l = jnp.exp(sl_ref[...])
    E_lm1 = jnp.exp(slm1_ref[...])
    E79 = jnp.exp(s79_ref[...])
    E80 = jnp.exp(s80_ref[...])
    E81 = jnp.exp(s81_ref[...])

    A = _digamma_large(T - E_l - 1.0)
    B = _digamma_large(T - E80 - 1.0)
    le79 = lab <= 79
    E_t = jnp.where(le79, E80, E79)
    E_g = jnp.where(le79, E_l, E81)
    ff = (1.0 - obj) * (A - _digamma_small(E_t + 1.0)) + jnp.where(
        lab != 81, obj * (B - _digamma_small(E_g + 1.0)), 0.0)
    x_t = jnp.where(lab == 81, E80, E81)
    x_g = jnp.where((lab >= 1) & (lab <= 80), E_lm1, E81)
    fb = obj * (A - _digamma_small(x_t + 1.0)) + 0.2 * (1.0 - obj) * (
        B - _digamma_small(x_g + 1.0))

    kp = kp_ref[...]
    kn = kn_ref[...]
    r = jax.lax.broadcasted_iota(jnp.uint32, kp.shape, 0)
    c = jax.lax.broadcasted_iota(jnp.uint32, kp.shape, 1)
    ik = jnp.uint32(_N - 1) - (r * jnp.uint32(kp.shape[1]) + c)
    s_pos = _select_sum(kp, ik, ff, _K_POS)
    s_neg = _select_sum(kn, ik, fb, _K_NEG)
    out_ref[...] = jnp.full((1, 1), (s_pos + s_neg) / float(_K_POS + _K_NEG))


def kernel(scores, labels, squarescores, objectness, ious):
    del squarescores, ious
    lab2 = labels.reshape(_N, 1)
    nblk = _N // _BLK
    col = jax.ShapeDtypeStruct((_N, 1), jnp.float32)
    colu = jax.ShapeDtypeStruct((_N, 1), jnp.uint32)
    cspec = pl.BlockSpec((_BLK, 1), lambda i: (i, 0))
    outs = pl.pallas_call(
        _pass1_body,
        grid=(nblk,),
        in_specs=[
            pl.BlockSpec((_BLK, _C), lambda i: (i, 0)),
            cspec,
        ],
        out_specs=[cspec] * 8,
        out_shape=[colu, colu, col, col, col, col, col, col],
        interpret=_INTERPRET,
    )(scores, lab2)

    shape2 = (_N // 128, 128)
    dense = [x.reshape(shape2) for x in outs]
    dense.append(labels.reshape(shape2))
    dense.append(objectness.reshape(shape2))
    out = pl.pallas_call(
        _pass2_body,
        out_shape=jax.ShapeDtypeStruct((1, 1), jnp.float32),
        interpret=_INTERPRET,
    )(*dense)
    return out[0, 0]


# speedup vs baseline: 2.8112x; 1.6831x over previous
import jax
import jax.numpy as jnp
from jax.experimental import pallas as pl

_N = 65536
_C = 82
_K_POS = 256
_K_NEG = 512
_BLK = 1024
_INTERPRET = False


def _digamma_large(x):
    inv = 1.0 / x
    inv2 = inv * inv
    return jnp.log(x) - 0.5 * inv - inv2 * (
        (1.0 / 12.0) - inv2 * ((1.0 / 120.0) - inv2 * (1.0 / 252.0)))


def _digamma_small(x):
    q = ((((x + 15.0) * x + 85.0) * x + 225.0) * x + 274.0) * x * x + 120.0 * x
    qp = ((((6.0 * x + 75.0) * x + 340.0) * x + 675.0) * x + 548.0) * x + 120.0
    return _digamma_large(x + 6.0) - qp / q


def _sort_key(x):
    u = jax.lax.bitcast_convert_type(x, jnp.uint32)
    sign = u >> jnp.uint32(31)
    flip = sign * jnp.uint32(0x7FFFFFFF) + jnp.uint32(0x80000000)
    return u ^ flip


def _pack(x):
    return x.reshape(_BLK // 128, 128)


def _pass1_body(scores_ref, labels_ref, kp_ref, kn_ref, t_ref, sl_ref,
                slm1_ref, s79_ref, s80_ref, s81_ref):
    s = scores_ref[...]
    lab = labels_ref[...]
    E = jnp.exp(s)
    t_ref[...] = _pack(jnp.sum(E, axis=1, keepdims=True))
    cols = jax.lax.broadcasted_iota(jnp.int32, s.shape, 1)
    sl_ref[...] = _pack(jnp.sum(jnp.where(cols == lab, s, 0.0), axis=1,
                                keepdims=True))
    slm1_ref[...] = _pack(jnp.sum(jnp.where(cols == lab - 1, s, 0.0), axis=1,
                                  keepdims=True))
    s79_ref[...] = _pack(s[:, 79:80])
    s80_ref[...] = _pack(s[:, 80:81])
    s81_ref[...] = _pack(s[:, 81:82])
    m80 = jnp.max(s[:, :80], axis=1, keepdims=True)
    fg = lab != 81
    pos = jnp.where(fg, -m80, -jnp.inf)
    neg = jnp.where(fg, -jnp.inf, -s[:, 81:82])
    kp_ref[...] = _pack(_sort_key(pos))
    kn_ref[...] = _pack(_sort_key(neg))


def _select_sum(keys, ik, vals, k):
    def hi_body(i, a):
        b = (jnp.int32(31) - i).astype(jnp.uint32)
        trial = a | (jnp.uint32(1) << b)
        cnt = jnp.sum((keys >= trial).astype(jnp.int32))
        return jnp.where(cnt >= k, trial, a)

    a_hi = jax.lax.fori_loop(0, 32, hi_body, jnp.uint32(0))
    gt = keys > a_hi
    eq = keys == a_hi
    cnt_gt = jnp.sum(gt.astype(jnp.int32))

    def lo_body(i, a):
        b = (jnp.int32(15) - i).astype(jnp.uint32)
        trial = a | (jnp.uint32(1) << b)
        cnt = cnt_gt + jnp.sum((eq & (ik >= trial)).astype(jnp.int32))
        return jnp.where(cnt >= k, trial, a)

    a_lo = jax.lax.fori_loop(0, 16, lo_body, jnp.uint32(0))
    sel = gt | (eq & (ik >= a_lo))
    return jnp.sum(jnp.where(sel, vals, 0.0))


def _pass2_body(kp_ref, kn_ref, t_ref, sl_ref, slm1_ref, s79_ref, s80_ref,
                s81_ref, lab_ref, obj_ref, out_ref):
    lab = lab_ref[...]
    obj = obj_ref[...]
    T = t_ref[...] + float(_C)
    E_l = jnp.exp(sl_ref[...])
    E_lm1 = jnp.exp(slm1_ref[...])
    E79 = jnp.exp(s79_ref[...])
    E80 = jnp.exp(s80_ref[...])
    E81 = jnp.exp(s81_ref[...])

    A = _digamma_large(T - E_l - 1.0)
    B = _digamma_large(T - E80 - 1.0)
    le79 = lab <= 79
    E_t = jnp.where(le79, E80, E79)
    E_g = jnp.where(le79, E_l, E81)
    ff = (1.0 - obj) * (A - _digamma_small(E_t + 1.0)) + jnp.where(
        lab != 81, obj * (B - _digamma_small(E_g + 1.0)), 0.0)
    x_t = jnp.where(lab == 81, E80, E81)
    x_g = jnp.where((lab >= 1) & (lab <= 80), E_lm1, E81)
    fb = obj * (A - _digamma_small(x_t + 1.0)) + 0.2 * (1.0 - obj) * (
        B - _digamma_small(x_g + 1.0))

    kp = kp_ref[...]
    kn = kn_ref[...]
    r = jax.lax.broadcasted_iota(jnp.uint32, kp.shape, 0)
    c = jax.lax.broadcasted_iota(jnp.uint32, kp.shape, 1)
    ik = jnp.uint32(_N - 1) - (r * jnp.uint32(kp.shape[1]) + c)
    s_pos = _select_sum(kp, ik, ff, _K_POS)
    s_neg = _select_sum(kn, ik, fb, _K_NEG)
    out_ref[...] = jnp.full((1, 1), (s_pos + s_neg) / float(_K_POS + _K_NEG))


def kernel(scores, labels, squarescores, objectness, ious):
    del squarescores, ious
    lab2 = labels.reshape(_N, 1)
    nblk = _N // _BLK
    shape2 = (_N // 128, 128)
    col = jax.ShapeDtypeStruct(shape2, jnp.float32)
    colu = jax.ShapeDtypeStruct(shape2, jnp.uint32)
    cspec = pl.BlockSpec((_BLK // 128, 128), lambda i: (i, 0))
    outs = pl.pallas_call(
        _pass1_body,
        grid=(nblk,),
        in_specs=[
            pl.BlockSpec((_BLK, _C), lambda i: (i, 0)),
            pl.BlockSpec((_BLK, 1), lambda i: (i, 0)),
        ],
        out_specs=[cspec] * 8,
        out_shape=[colu, colu, col, col, col, col, col, col],
        interpret=_INTERPRET,
    )(scores, lab2)

    dense = list(outs)
    dense.append(labels.reshape(shape2))
    dense.append(objectness.reshape(shape2))
    out = pl.pallas_call(
        _pass2_body,
        out_shape=jax.ShapeDtypeStruct((1, 1), jnp.float32),
        interpret=_INTERPRET,
    )(*dense)
    return out[0, 0]


# speedup vs baseline: 3.2648x; 1.1613x over previous
import jax
import jax.numpy as jnp
from jax.experimental import pallas as pl

_N = 65536
_C = 82
_K_POS = 256
_K_NEG = 512
_BLK = 1024
_INTERPRET = False


def _digamma_large(x):
    inv = 1.0 / x
    inv2 = inv * inv
    return jnp.log(x) - 0.5 * inv - inv2 * (
        (1.0 / 12.0) - inv2 * ((1.0 / 120.0) - inv2 * (1.0 / 252.0)))


def _digamma_small(x):
    q = ((((x + 15.0) * x + 85.0) * x + 225.0) * x + 274.0) * x * x + 120.0 * x
    qp = ((((6.0 * x + 75.0) * x + 340.0) * x + 675.0) * x + 548.0) * x + 120.0
    return _digamma_large(x + 6.0) - qp / q


def _sort_key(x):
    u = jax.lax.bitcast_convert_type(x, jnp.uint32)
    sign = u >> jnp.uint32(31)
    flip = sign * jnp.uint32(0x7FFFFFFF) + jnp.uint32(0x80000000)
    return u ^ flip


def _pack(x):
    return x.reshape(_BLK // 128, 128)


def _pass1_body(scores_ref, labels_ref, m80_ref, t_ref, sl_ref,
                slm1_ref, s79_ref, s80_ref, s81_ref):
    s = scores_ref[...]
    lab = labels_ref[...]
    E = jnp.exp(s)
    ones = jnp.ones((_C, 1), dtype=jnp.float32)
    t_ref[...] = _pack(jax.lax.dot(E, ones))
    cols = jax.lax.broadcasted_iota(jnp.int32, s.shape, 1)
    sl_ref[...] = _pack(jax.lax.dot(jnp.where(cols == lab, s, 0.0), ones))
    slm1_ref[...] = _pack(jax.lax.dot(jnp.where(cols == lab - 1, s, 0.0),
                                      ones))
    s79_ref[...] = _pack(s[:, 79:80])
    s80_ref[...] = _pack(s[:, 80:81])
    s81_ref[...] = _pack(s[:, 81:82])
    sm = jnp.where(cols < 80, s, -jnp.inf)
    m80_ref[...] = _pack(jnp.max(sm, axis=1, keepdims=True))


def _select_sum(keys, ik, vals, k):
    def hi_body(i, a):
        b = (jnp.int32(31) - i).astype(jnp.uint32)
        trial = a | (jnp.uint32(1) << b)
        cnt = jnp.sum((keys >= trial).astype(jnp.int32))
        return jnp.where(cnt >= k, trial, a)

    a_hi = jax.lax.fori_loop(0, 32, hi_body, jnp.uint32(0))
    gt = keys > a_hi
    eq = keys == a_hi
    cnt_gt = jnp.sum(gt.astype(jnp.int32))

    def lo_body(i, a):
        b = (jnp.int32(15) - i).astype(jnp.uint32)
        trial = a | (jnp.uint32(1) << b)
        cnt = cnt_gt + jnp.sum((eq & (ik >= trial)).astype(jnp.int32))
        return jnp.where(cnt >= k, trial, a)

    a_lo = jax.lax.fori_loop(0, 16, lo_body, jnp.uint32(0))
    sel = gt | (eq & (ik >= a_lo))
    return jnp.sum(jnp.where(sel, vals, 0.0))


def _pass2_body(m80_ref, t_ref, sl_ref, slm1_ref, s79_ref, s80_ref,
                s81_ref, lab_ref, obj_ref, out_ref):
    lab = lab_ref[...]
    obj = obj_ref[...]
    T = t_ref[...] + float(_C)
    E_l = jnp.exp(sl_ref[...])
    E_lm1 = jnp.exp(slm1_ref[...])
    E79 = jnp.exp(s79_ref[...])
    E80 = jnp.exp(s80_ref[...])
    E81 = jnp.exp(s81_ref[...])

    A = _digamma_large(T - E_l - 1.0)
    B = _digamma_large(T - E80 - 1.0)
    le79 = lab <= 79
    E_t = jnp.where(le79, E80, E79)
    E_g = jnp.where(le79, E_l, E81)
    ff = (1.0 - obj) * (A - _digamma_small(E_t + 1.0)) + jnp.where(
        lab != 81, obj * (B - _digamma_small(E_g + 1.0)), 0.0)
    x_t = jnp.where(lab == 81, E80, E81)
    x_g = jnp.where((lab >= 1) & (lab <= 80), E_lm1, E81)
    fb = obj * (A - _digamma_small(x_t + 1.0)) + 0.2 * (1.0 - obj) * (
        B - _digamma_small(x_g + 1.0))

    fg = lab != 81
    kp = _sort_key(jnp.where(fg, -m80_ref[...], -jnp.inf))
    kn = _sort_key(jnp.where(fg, -jnp.inf, -s81_ref[...]))
    r = jax.lax.broadcasted_iota(jnp.uint32, kp.shape, 0)
    c = jax.lax.broadcasted_iota(jnp.uint32, kp.shape, 1)
    ik = jnp.uint32(_N - 1) - (r * jnp.uint32(kp.shape[1]) + c)
    s_pos = _select_sum(kp, ik, ff, _K_POS)
    s_neg = _select_sum(kn, ik, fb, _K_NEG)
    out_ref[...] = jnp.full((1, 1), (s_pos + s_neg) / float(_K_POS + _K_NEG))


def kernel(scores, labels, squarescores, objectness, ious):
    del squarescores, ious
    nblk = _N // _BLK
    shape2 = (_N // 128, 128)
    lab2 = labels.reshape(_N, 1)
    col = jax.ShapeDtypeStruct(shape2, jnp.float32)
    cspec = pl.BlockSpec((_BLK // 128, 128), lambda i: (i, 0))
    outs = pl.pallas_call(
        _pass1_body,
        grid=(nblk,),
        in_specs=[
            pl.BlockSpec((_BLK, _C), lambda i: (i, 0)),
            pl.BlockSpec((_BLK, 1), lambda i: (i, 0)),
        ],
        out_specs=[cspec] * 7,
        out_shape=[col] * 7,
        interpret=_INTERPRET,
    )(scores, lab2)

    dense = list(outs)
    dense.append(labels.reshape(shape2))
    dense.append(objectness.reshape(shape2))
    out = pl.pallas_call(
        _pass2_body,
        out_shape=jax.ShapeDtypeStruct((1, 1), jnp.float32),
        interpret=_INTERPRET,
    )(*dense)
    return out[0, 0]
